# Initial kernel scaffold; baseline (speedup 1.0000x reference)
import jax
import jax.numpy as jnp
from jax.experimental import pallas as pl


def kernel(x, edge_index, W, b):
    raise NotImplementedError("write your pallas kernel here")


# speedup vs baseline: 15.2773x; 15.2773x over previous
import functools

import jax
import jax.numpy as jnp
from jax import lax
from jax.experimental import pallas as pl
from jax.experimental.pallas import tpu as pltpu
from jax.experimental.pallas import tpu_sc as plsc

NC = 2
NS = 16
LANES = 16
CHUNK = 128


def _sc_mesh():
    return plsc.VectorSubcoreMesh(core_axis_name="c", subcore_axis_name="s")


def _sc_degree(dst_pad, n_pad, e_tile, rows_per_tile):
    n_chunks = e_tile // CHUNK

    @functools.partial(
        pl.kernel,
        out_type=jax.ShapeDtypeStruct((NC, n_pad), jnp.float32),
        mesh=_sc_mesh(),
        scratch_types=[
            pltpu.VMEM((CHUNK,), jnp.int32),
            pltpu.VMEM((CHUNK,), jnp.float32),
            pltpu.VMEM((rows_per_tile,), jnp.float32),
            pltpu.VMEM_SHARED((n_pad,), jnp.float32),
        ],
    )
    def k(dst_hbm, out_hbm, idx_v, ones_v, drain_v, deg_shared):
        c = lax.axis_index("c")
        s = lax.axis_index("s")
        row0 = pl.multiple_of(s * rows_per_tile, CHUNK)

        @pl.loop(0, rows_per_tile // LANES)
        def _zero(i):
            drain_v[pl.ds(i * LANES, LANES)] = jnp.zeros((LANES,), jnp.float32)

        pltpu.sync_copy(drain_v, deg_shared.at[pl.ds(row0, rows_per_tile)])
        for j in range(CHUNK // LANES):
            ones_v[pl.ds(j * LANES, LANES)] = jnp.ones((LANES,), jnp.float32)
        plsc.subcore_barrier()

        base0 = (c * NS + s) * e_tile

        @pl.loop(0, n_chunks)
        def _acc(i):
            base = pl.multiple_of(base0 + i * CHUNK, CHUNK)
            pltpu.sync_copy(dst_hbm.at[pl.ds(base, CHUNK)], idx_v)
            pltpu.sync_copy(ones_v, deg_shared.at[idx_v], add=True)

        plsc.subcore_barrier()
        pltpu.sync_copy(deg_shared.at[pl.ds(row0, rows_per_tile)], drain_v)
        pltpu.sync_copy(drain_v, out_hbm.at[c, pl.ds(row0, rows_per_tile)])

    return k(dst_pad)


def _sc_aggregate(g, src_pad, dst_pad, n_pad, e_tile, rows_per_tile):
    n, d = g.shape
    n_chunks = e_tile // CHUNK
    drain_chunks = rows_per_tile // CHUNK

    @functools.partial(
        pl.kernel,
        out_type=jax.ShapeDtypeStruct((NC, n_pad, d), jnp.float32),
        mesh=_sc_mesh(),
        scratch_types=[
            pltpu.VMEM((CHUNK,), jnp.int32),
            pltpu.VMEM((CHUNK,), jnp.int32),
            pltpu.VMEM((CHUNK, d), jnp.float32),
            pltpu.VMEM_SHARED((n_pad, d), jnp.float32),
            pltpu.SemaphoreType.DMA,
        ],
    )
    def k(g_hbm, src_hbm, dst_hbm, out_hbm, src_v, dst_v, rows_v, acc_shared, sem):
        c = lax.axis_index("c")
        s = lax.axis_index("s")
        row0 = pl.multiple_of(s * rows_per_tile, CHUNK)

        @pl.loop(0, CHUNK)
        def _zero(i):
            for j in range(d // LANES):
                rows_v[i, pl.ds(j * LANES, LANES)] = jnp.zeros((LANES,), jnp.float32)

        @pl.loop(0, drain_chunks)
        def _zacc(kk):
            r = pl.multiple_of(row0 + kk * CHUNK, CHUNK)
            pltpu.sync_copy(rows_v, acc_shared.at[pl.ds(r, CHUNK)])

        plsc.subcore_barrier()

        base0 = (c * NS + s) * e_tile

        @pl.loop(0, n_chunks)
        def _acc(i):
            base = pl.multiple_of(base0 + i * CHUNK, CHUNK)
            pltpu.sync_copy(src_hbm.at[pl.ds(base, CHUNK)], src_v)
            pltpu.sync_copy(dst_hbm.at[pl.ds(base, CHUNK)], dst_v)
            pltpu.async_copy(g_hbm.at[src_v], rows_v, sem).wait()
            pltpu.sync_copy(rows_v, acc_shared.at[dst_v], add=True)

        plsc.subcore_barrier()

        @pl.loop(0, drain_chunks)
        def _drain(kk):
            r = pl.multiple_of(row0 + kk * CHUNK, CHUNK)
            pltpu.sync_copy(acc_shared.at[pl.ds(r, CHUNK)], rows_v)
            pltpu.sync_copy(rows_v, out_hbm.at[c, pl.ds(r, CHUNK)])

    return k(g, src_pad, dst_pad)


def _tc_linear(x, w, deg2):
    n, d_in = x.shape
    d_out = w.shape[1]
    blk = 1000

    def body(x_ref, w_ref, deg_ref, g_ref, dis_ref):
        deg = deg_ref[:, 0:1] + deg_ref[:, 1:2] + 1.0
        dis = lax.rsqrt(deg)
        h = jnp.dot(x_ref[...], w_ref[...], preferred_element_type=jnp.float32)
        g_ref[...] = h * dis
        dis_ref[...] = dis

    return pl.pallas_call(
        body,
        grid=(n // blk,),
        in_specs=[
            pl.BlockSpec((blk, d_in), lambda i: (i, 0)),
            pl.BlockSpec((d_in, d_out), lambda i: (0, 0)),
            pl.BlockSpec((blk, 2), lambda i: (i, 0)),
        ],
        out_specs=[
            pl.BlockSpec((blk, d_out), lambda i: (i, 0)),
            pl.BlockSpec((blk, 1), lambda i: (i, 0)),
        ],
        out_shape=[
            jax.ShapeDtypeStruct((n, d_out), jnp.float32),
            jax.ShapeDtypeStruct((n, 1), jnp.float32),
        ],
    )(x, w, deg2)


def _tc_combine(acc, g, dis, b2):
    n, d = g.shape
    blk = 1000

    def body(acc_ref, g_ref, dis_ref, b_ref, out_ref):
        total = acc_ref[0] + acc_ref[1] + g_ref[...]
        out_ref[...] = dis_ref[...] * total + b_ref[...]

    return pl.pallas_call(
        body,
        grid=(n // blk,),
        in_specs=[
            pl.BlockSpec((2, blk, d), lambda i: (0, i, 0)),
            pl.BlockSpec((blk, d), lambda i: (i, 0)),
            pl.BlockSpec((blk, 1), lambda i: (i, 0)),
            pl.BlockSpec((1, d), lambda i: (0, 0)),
        ],
        out_specs=pl.BlockSpec((blk, d), lambda i: (i, 0)),
        out_shape=jax.ShapeDtypeStruct((n, d), jnp.float32),
    )(acc, g, dis, b2)


def kernel(x, edge_index, W, b):
    n, d_in = x.shape
    d_out = W.shape[1]
    e = edge_index.shape[1]
    nw = NC * NS

    e_tile = pl.cdiv(e, nw * CHUNK) * CHUNK
    pad = nw * e_tile - e
    rows_per_tile = pl.cdiv(n + 1, NS * CHUNK) * CHUNK
    n_pad = NS * rows_per_tile

    src = edge_index[0]
    dst = edge_index[1]
    if pad:
        src = jnp.concatenate([src, jnp.zeros((pad,), jnp.int32)])
        dst = jnp.concatenate([dst, jnp.full((pad,), n, jnp.int32)])

    degs = _sc_degree(dst, n_pad, e_tile, rows_per_tile)
    deg2 = degs.T[:n]
    g, dis = _tc_linear(x, W, deg2)
    acc = _sc_aggregate(g, src, dst, n_pad, e_tile, rows_per_tile)
    out = _tc_combine(acc[:, :n], g, dis, b.reshape(1, d_out))
    return out


# speedup vs baseline: 17.5899x; 1.1514x over previous
import functools

import jax
import jax.numpy as jnp
from jax import lax
from jax.experimental import pallas as pl
from jax.experimental.pallas import tpu as pltpu
from jax.experimental.pallas import tpu_sc as plsc

NC = 2
NS = 16
LANES = 16
CHUNK = 128
NIDX = 4


def _sc_mesh():
    return plsc.VectorSubcoreMesh(core_axis_name="c", subcore_axis_name="s")


def _sc_degree(dst_tiles, n_pad, rows_per_tile):
    n_chunks = dst_tiles.shape[1]

    @functools.partial(
        pl.kernel,
        out_type=jax.ShapeDtypeStruct((NC, n_pad), jnp.float32),
        mesh=_sc_mesh(),
        scratch_types=[
            pltpu.VMEM((n_chunks, CHUNK), jnp.int32),
            pltpu.VMEM((CHUNK,), jnp.float32),
            pltpu.VMEM((rows_per_tile,), jnp.float32),
            pltpu.VMEM_SHARED((n_pad,), jnp.float32),
            pltpu.SemaphoreType.DMA,
        ],
    )
    def k(dst_hbm, out_hbm, dst_all, ones_v, drain_v, deg_shared, sem):
        c = lax.axis_index("c")
        s = lax.axis_index("s")
        w = c * NS + s
        row0 = pl.multiple_of(s * rows_per_tile, CHUNK)

        pltpu.sync_copy(dst_hbm.at[w], dst_all)

        @pl.loop(0, rows_per_tile // LANES)
        def _zero(i):
            drain_v[pl.ds(i * LANES, LANES)] = jnp.zeros((LANES,), jnp.float32)

        pltpu.sync_copy(drain_v, deg_shared.at[pl.ds(row0, rows_per_tile)])
        for j in range(CHUNK // LANES):
            ones_v[pl.ds(j * LANES, LANES)] = jnp.ones((LANES,), jnp.float32)
        plsc.subcore_barrier()

        @pl.loop(0, n_chunks)
        def _fire(i):
            pltpu.async_copy(ones_v, deg_shared.at[dst_all.at[i]], sem, add=True)

        @pl.loop(0, n_chunks)
        def _drain(i):
            pltpu.make_async_copy(ones_v, deg_shared.at[dst_all.at[i]], sem).wait()

        plsc.subcore_barrier()
        pltpu.sync_copy(deg_shared.at[pl.ds(row0, rows_per_tile)], drain_v)
        pltpu.sync_copy(drain_v, out_hbm.at[c, pl.ds(row0, rows_per_tile)])

    return k(dst_tiles)


def _sc_aggregate(g, idx2, n_pad, rows_per_tile):
    n, d = g.shape
    n_chunks = idx2.shape[1]
    n_pairs = n_chunks // 2
    drain_chunks = rows_per_tile // CHUNK

    @functools.partial(
        pl.kernel,
        out_type=jax.ShapeDtypeStruct((NC, n_pad, d), jnp.float32),
        mesh=_sc_mesh(),
        scratch_types=[
            pltpu.VMEM((NIDX, 2, CHUNK), jnp.int32),
            pltpu.VMEM((2, CHUNK, d), jnp.float32),
            pltpu.VMEM_SHARED((n_pad, d), jnp.float32),
        ]
        + [pltpu.SemaphoreType.DMA] * (NIDX + 4),
    )
    def k(g_hbm, idx_hbm, out_hbm, sd, rows, acc_shared, *sems):
        isem = sems[:NIDX]
        gsem = sems[NIDX:NIDX + 2]
        ssem = sems[NIDX + 2:]
        c = lax.axis_index("c")
        s = lax.axis_index("s")
        w = c * NS + s
        row0 = pl.multiple_of(s * rows_per_tile, CHUNK)

        def i_start(i, q):
            pltpu.async_copy(idx_hbm.at[w, i], sd.at[q], isem[q])

        def i_wait(i, q):
            pltpu.make_async_copy(idx_hbm.at[w, i], sd.at[q], isem[q]).wait()

        def g_start(q, b):
            pltpu.async_copy(g_hbm.at[sd.at[q, 0]], rows.at[b], gsem[b])

        def g_wait(q, b):
            pltpu.make_async_copy(g_hbm.at[sd.at[q, 0]], rows.at[b], gsem[b]).wait()

        def s_start(q, b):
            pltpu.async_copy(rows.at[b], acc_shared.at[sd.at[q, 1]], ssem[b], add=True)

        def s_wait(q, b):
            pltpu.make_async_copy(rows.at[b], acc_shared.at[sd.at[q, 1]], ssem[b]).wait()

        @pl.loop(0, CHUNK)
        def _zrows(i):
            for j in range(d // LANES):
                rows[0, i, pl.ds(j * LANES, LANES)] = jnp.zeros((LANES,), jnp.float32)

        @pl.loop(0, drain_chunks)
        def _zacc(kk):
            r = pl.multiple_of(row0 + kk * CHUNK, CHUNK)
            pltpu.sync_copy(rows.at[0], acc_shared.at[pl.ds(r, CHUNK)])

        plsc.subcore_barrier()

        for q in range(2):
            i_start(q, q)
        for q in range(2):
            i_wait(q, q)
            g_start(q, q)

        @pl.loop(0, n_chunks // 4)
        def _pipe(t):
            for k in range(4):
                i = t * 4 + k
                q = k
                qn = (k + 2) % 4
                bslot = k % 2

                @pl.when(i + 2 < n_chunks)
                def _pre():
                    i_start(i + 2, qn)

                g_wait(q, bslot)
                s_start(q, bslot)

                @pl.when(i + 2 < n_chunks)
                def _nxt():
                    s_wait(q, bslot)
                    i_wait(i + 2, qn)
                    g_start(qn, bslot)

        s_wait(2, 0)
        s_wait(3, 1)

        plsc.subcore_barrier()

        @pl.loop(0, drain_chunks)
        def _drain(kk):
            r = pl.multiple_of(row0 + kk * CHUNK, CHUNK)
            pltpu.sync_copy(acc_shared.at[pl.ds(r, CHUNK)], rows.at[0])
            pltpu.sync_copy(rows.at[0], out_hbm.at[c, pl.ds(r, CHUNK)])

    return k(g, idx2)


def _tc_linear(x, w, deg2):
    n, d_in = x.shape
    d_out = w.shape[1]
    blk = 1000

    def body(x_ref, w_ref, deg_ref, g_ref, dis_ref):
        deg = deg_ref[:, 0:1] + deg_ref[:, 1:2] + 1.0
        dis = lax.rsqrt(deg)
        h = jnp.dot(x_ref[...], w_ref[...], preferred_element_type=jnp.float32)
        g_ref[...] = h * dis
        dis_ref[...] = dis

    return pl.pallas_call(
        body,
        grid=(n // blk,),
        in_specs=[
            pl.BlockSpec((blk, d_in), lambda i: (i, 0)),
            pl.BlockSpec((d_in, d_out), lambda i: (0, 0)),
            pl.BlockSpec((blk, 2), lambda i: (i, 0)),
        ],
        out_specs=[
            pl.BlockSpec((blk, d_out), lambda i: (i, 0)),
            pl.BlockSpec((blk, 1), lambda i: (i, 0)),
        ],
        out_shape=[
            jax.ShapeDtypeStruct((n, d_out), jnp.float32),
            jax.ShapeDtypeStruct((n, 1), jnp.float32),
        ],
    )(x, w, deg2)


def _tc_combine(acc, g, dis, b2):
    n, d = g.shape
    blk = 1000

    def body(acc_ref, g_ref, dis_ref, b_ref, out_ref):
        total = acc_ref[0] + acc_ref[1] + g_ref[...]
        out_ref[...] = dis_ref[...] * total + b_ref[...]

    return pl.pallas_call(
        body,
        grid=(n // blk,),
        in_specs=[
            pl.BlockSpec((2, blk, d), lambda i: (0, i, 0)),
            pl.BlockSpec((blk, d), lambda i: (i, 0)),
            pl.BlockSpec((blk, 1), lambda i: (i, 0)),
            pl.BlockSpec((1, d), lambda i: (0, 0)),
        ],
        out_specs=pl.BlockSpec((blk, d), lambda i: (i, 0)),
        out_shape=jax.ShapeDtypeStruct((n, d), jnp.float32),
    )(acc, g, dis, b2)


def kernel(x, edge_index, W, b):
    n, d_in = x.shape
    d_out = W.shape[1]
    e = edge_index.shape[1]
    nw = NC * NS

    rows_per_tile = pl.cdiv(n + 1, NS * CHUNK) * CHUNK
    n_pad = NS * rows_per_tile

    src = edge_index[0]
    dst = edge_index[1]

    e_tile1 = pl.cdiv(e, nw * CHUNK) * CHUNK
    pad1 = nw * e_tile1 - e
    dst1 = jnp.concatenate([dst, jnp.full((pad1,), n, jnp.int32)])
    dst1 = dst1.reshape(nw, e_tile1 // CHUNK, CHUNK)

    degs = _sc_degree(dst1, n_pad, rows_per_tile)
    deg2 = degs.T[:n]
    g, dis = _tc_linear(x, W, deg2)

    e_tile3 = pl.cdiv(e, nw * 2 * NIDX * CHUNK) * 2 * NIDX * CHUNK
    pad3 = nw * e_tile3 - e
    src3 = jnp.concatenate([src, jnp.zeros((pad3,), jnp.int32)])
    dst3 = jnp.concatenate([dst, jnp.full((pad3,), n, jnp.int32)])
    nch = e_tile3 // CHUNK
    idx2 = jnp.stack(
        [src3.reshape(nw, nch, CHUNK), dst3.reshape(nw, nch, CHUNK)], axis=2
    )

    acc = _sc_aggregate(g, idx2, n_pad, rows_per_tile)
    out = _tc_combine(acc[:, :n], g, dis, b.reshape(1, d_out))
    return out


# speedup vs baseline: 46.1889x; 2.6259x over previous
import functools

import jax
import jax.numpy as jnp
from jax import lax
from jax.experimental import pallas as pl
from jax.experimental.pallas import tpu as pltpu
from jax.experimental.pallas import tpu_sc as plsc

NC = 2
NS = 16
LANES = 16
CHUNK = 128
NIDX = 4


def _sc_mesh():
    return plsc.VectorSubcoreMesh(core_axis_name="c", subcore_axis_name="s")


def _sc_degree(dst_tiles, n_pad, rows_per_tile):
    n_chunks = dst_tiles.shape[1]

    @functools.partial(
        pl.kernel,
        out_type=jax.ShapeDtypeStruct((NC, n_pad), jnp.float32),
        mesh=_sc_mesh(),
        scratch_types=[
            pltpu.VMEM((n_chunks, CHUNK), jnp.int32),
            pltpu.VMEM((CHUNK,), jnp.float32),
            pltpu.VMEM((rows_per_tile,), jnp.float32),
            pltpu.VMEM_SHARED((n_pad,), jnp.float32),
            pltpu.SemaphoreType.DMA,
        ],
    )
    def k(dst_hbm, out_hbm, dst_all, ones_v, drain_v, deg_shared, sem):
        c = lax.axis_index("c")
        s = lax.axis_index("s")
        w = c * NS + s
        row0 = pl.multiple_of(s * rows_per_tile, CHUNK)

        pltpu.sync_copy(dst_hbm.at[w], dst_all)

        @pl.loop(0, rows_per_tile // LANES)
        def _zero(i):
            drain_v[pl.ds(i * LANES, LANES)] = jnp.zeros((LANES,), jnp.float32)

        pltpu.sync_copy(drain_v, deg_shared.at[pl.ds(row0, rows_per_tile)])
        for j in range(CHUNK // LANES):
            ones_v[pl.ds(j * LANES, LANES)] = jnp.ones((LANES,), jnp.float32)
        plsc.subcore_barrier()

        @pl.loop(0, n_chunks)
        def _fire(i):
            pltpu.async_copy(ones_v, deg_shared.at[dst_all.at[i]], sem, add=True)

        @pl.loop(0, n_chunks)
        def _drain(i):
            pltpu.make_async_copy(ones_v, deg_shared.at[dst_all.at[i]], sem).wait()

        plsc.subcore_barrier()
        pltpu.sync_copy(deg_shared.at[pl.ds(row0, rows_per_tile)], drain_v)
        pltpu.sync_copy(drain_v, out_hbm.at[c, pl.ds(row0, rows_per_tile)])

    return k(dst_tiles)


def _sc_aggregate(g, idx2, n_pad, rows_per_tile):
    n, d = g.shape
    n_chunks = idx2.shape[1]
    n_pairs = n_chunks // 2
    drain_chunks = rows_per_tile // CHUNK

    @functools.partial(
        pl.kernel,
        out_type=jax.ShapeDtypeStruct((NC, n_pad, d), jnp.float32),
        mesh=_sc_mesh(),
        scratch_types=[
            pltpu.VMEM((NIDX, 2, CHUNK), jnp.int32),
            pltpu.VMEM((2, CHUNK, d), jnp.float32),
            pltpu.VMEM_SHARED((n_pad, d), jnp.float32),
        ]
        + [pltpu.SemaphoreType.DMA] * (NIDX + 4),
    )
    def k(g_hbm, idx_hbm, out_hbm, sd, rows, acc_shared, *sems):
        isem = sems[:NIDX]
        gsem = sems[NIDX:NIDX + 2]
        ssem = sems[NIDX + 2:]
        c = lax.axis_index("c")
        s = lax.axis_index("s")
        w = c * NS + s
        row0 = pl.multiple_of(s * rows_per_tile, CHUNK)

        def i_start(i, q):
            pltpu.async_copy(idx_hbm.at[w, i], sd.at[q], isem[q])

        def i_wait(i, q):
            pltpu.make_async_copy(idx_hbm.at[w, i], sd.at[q], isem[q]).wait()

        def g_start(q, b):
            pltpu.async_copy(g_hbm.at[sd.at[q, 0]], rows.at[b], gsem[b])

        def g_wait(q, b):
            pltpu.make_async_copy(g_hbm.at[sd.at[q, 0]], rows.at[b], gsem[b]).wait()

        def s_start(q, b):
            pltpu.async_copy(rows.at[b], acc_shared.at[sd.at[q, 1]], ssem[b], add=True)

        def s_wait(q, b):
            pltpu.make_async_copy(rows.at[b], acc_shared.at[sd.at[q, 1]], ssem[b]).wait()

        @pl.loop(0, CHUNK)
        def _zrows(i):
            for j in range(d // LANES):
                rows[0, i, pl.ds(j * LANES, LANES)] = jnp.zeros((LANES,), jnp.float32)

        @pl.loop(0, drain_chunks)
        def _zacc(kk):
            r = pl.multiple_of(row0 + kk * CHUNK, CHUNK)
            pltpu.sync_copy(rows.at[0], acc_shared.at[pl.ds(r, CHUNK)])

        plsc.subcore_barrier()

        for q in range(2):
            i_start(q, q)
        for q in range(2):
            i_wait(q, q)
            g_start(q, q)

        @pl.loop(0, n_chunks // 4)
        def _pipe(t):
            for k in range(4):
                i = t * 4 + k
                q = k
                qn = (k + 2) % 4
                bslot = k % 2

                @pl.when(i + 2 < n_chunks)
                def _pre():
                    i_start(i + 2, qn)

                g_wait(q, bslot)
                s_start(q, bslot)

                @pl.when(i + 2 < n_chunks)
                def _nxt():
                    s_wait(q, bslot)
                    i_wait(i + 2, qn)
                    g_start(qn, bslot)

        s_wait(2, 0)
        s_wait(3, 1)

        plsc.subcore_barrier()

        @pl.loop(0, drain_chunks)
        def _drain(kk):
            r = pl.multiple_of(row0 + kk * CHUNK, CHUNK)
            pltpu.sync_copy(acc_shared.at[pl.ds(r, CHUNK)], rows.at[0])
            pltpu.sync_copy(rows.at[0], out_hbm.at[c, pl.ds(r, CHUNK)])

    return k(g, idx2)


def _tc_linear(x, w, deg2, blk):
    n, d_in = x.shape
    d_out = w.shape[1]

    def body(x_ref, w_ref, deg_ref, g_ref, dis_ref):
        deg = deg_ref[:, 0:1] + deg_ref[:, 1:2] + 1.0
        dis = lax.rsqrt(deg)
        h = jnp.dot(x_ref[...], w_ref[...], preferred_element_type=jnp.float32)
        g_ref[...] = h * dis
        dis_ref[...] = dis

    return pl.pallas_call(
        body,
        grid=(n // blk,),
        in_specs=[
            pl.BlockSpec((blk, d_in), lambda i: (i, 0)),
            pl.BlockSpec((d_in, d_out), lambda i: (0, 0)),
            pl.BlockSpec((blk, 2), lambda i: (i, 0)),
        ],
        out_specs=[
            pl.BlockSpec((blk, d_out), lambda i: (i, 0)),
            pl.BlockSpec((blk, 1), lambda i: (i, 0)),
        ],
        out_shape=[
            jax.ShapeDtypeStruct((n, d_out), jnp.float32),
            jax.ShapeDtypeStruct((n, 1), jnp.float32),
        ],
    )(x, w, deg2)


def _tc_combine(acc, g, dis, b2, n):
    d = g.shape[1]
    blk = 1000

    def body(acc_ref, g_ref, dis_ref, b_ref, out_ref):
        total = acc_ref[0] + acc_ref[1] + g_ref[...]
        out_ref[...] = dis_ref[...] * total + b_ref[...]

    return pl.pallas_call(
        body,
        grid=(n // blk,),
        in_specs=[
            pl.BlockSpec((2, blk, d), lambda i: (0, i, 0)),
            pl.BlockSpec((blk, d), lambda i: (i, 0)),
            pl.BlockSpec((blk, 1), lambda i: (i, 0)),
            pl.BlockSpec((1, d), lambda i: (0, 0)),
        ],
        out_specs=pl.BlockSpec((blk, d), lambda i: (i, 0)),
        out_shape=jax.ShapeDtypeStruct((n, d), jnp.float32),
    )(acc, g, dis, b2)


def kernel(x, edge_index, W, b):
    n, d_in = x.shape
    d_out = W.shape[1]
    e = edge_index.shape[1]
    nw = NC * NS

    rows_per_tile = pl.cdiv(n + 1, NS * CHUNK) * CHUNK
    n_pad = NS * rows_per_tile

    src = edge_index[0]
    dst = edge_index[1]

    e_tile1 = pl.cdiv(e, nw * CHUNK) * CHUNK
    pad1 = nw * e_tile1 - e
    trash1 = n + jnp.arange(pad1, dtype=jnp.int32) % (n_pad - n)
    dst1 = jnp.concatenate([dst, trash1])
    dst1 = dst1.reshape(nw, e_tile1 // CHUNK, CHUNK)

    degs = _sc_degree(dst1, n_pad, rows_per_tile)
    deg2 = degs.T

    x_pad = jnp.concatenate([x, jnp.zeros((n_pad - n, d_in), x.dtype)])
    g, dis = _tc_linear(x_pad, W, deg2, blk=n_pad // 10)

    e_tile3 = pl.cdiv(e, nw * 2 * NIDX * CHUNK) * 2 * NIDX * CHUNK
    pad3 = nw * e_tile3 - e
    zsrc = n + jnp.arange(pad3, dtype=jnp.int32) % (n_pad - n)
    zdst = jnp.arange(pad3, dtype=jnp.int32) % n_pad
    src3 = jnp.concatenate([src, zsrc])
    dst3 = jnp.concatenate([dst, zdst])
    nch = e_tile3 // CHUNK
    idx2 = jnp.stack(
        [src3.reshape(nw, nch, CHUNK), dst3.reshape(nw, nch, CHUNK)], axis=2
    )

    acc = _sc_aggregate(g, idx2, n_pad, rows_per_tile)
    out = _tc_combine(acc, g, dis, b.reshape(1, d_out), n)
    return out
